# baseline (speedup 1.0000x reference)
import jax
import jax.numpy as jnp
from jax.experimental import pallas as pl

_C, _H, _W = 3, 384, 384
_LANES = 1024
_ROWS = (_C * _H * _W) // _LANES
_BLOCK_ROWS = _ROWS // 2


def _equalize_block(x_ref, o_ref):
    o_ref[...] = x_ref[...].astype(jnp.uint8).astype(jnp.float32)


def kernel(pic):
    x = pic.reshape(_ROWS, _LANES)
    out = pl.pallas_call(
        _equalize_block,
        grid=(_ROWS // _BLOCK_ROWS,),
        in_specs=[pl.BlockSpec((_BLOCK_ROWS, _LANES), lambda i: (i, 0))],
        out_specs=pl.BlockSpec((_BLOCK_ROWS, _LANES), lambda i: (i, 0)),
        out_shape=jax.ShapeDtypeStruct((_ROWS, _LANES), jnp.float32),
    )(x)
    return out.reshape(_C, _H, _W)


# speedup vs baseline: 3.1431x; 3.1431x over previous
import jax
import jax.numpy as jnp
from jax.experimental import pallas as pl

_C, _H, _W = 3, 384, 384
_ROWS = _C * _H
_BLOCK_ROWS = 576


def _equalize_block(x_ref, o_ref):
    o_ref[...] = x_ref[...].astype(jnp.uint8).astype(jnp.float32)


def kernel(pic):
    x = pic.reshape(_ROWS, _W)
    out = pl.pallas_call(
        _equalize_block,
        grid=(_ROWS // _BLOCK_ROWS,),
        in_specs=[pl.BlockSpec((_BLOCK_ROWS, _W), lambda i: (i, 0))],
        out_specs=pl.BlockSpec((_BLOCK_ROWS, _W), lambda i: (i, 0)),
        out_shape=jax.ShapeDtypeStruct((_ROWS, _W), jnp.float32),
    )(x)
    return out.reshape(_C, _H, _W)


# speedup vs baseline: 3.1765x; 1.0106x over previous
import jax
import jax.numpy as jnp
from jax.experimental import pallas as pl

_C, _H, _W = 3, 384, 384
_ROWS = _C * _H
_BLOCK_ROWS = 576


def _equalize_block(x_ref, o_ref):
    o_ref[...] = (x_ref[...].astype(jnp.int32) & 255).astype(jnp.float32)


def kernel(pic):
    x = pic.reshape(_ROWS, _W)
    out = pl.pallas_call(
        _equalize_block,
        grid=(_ROWS // _BLOCK_ROWS,),
        in_specs=[pl.BlockSpec((_BLOCK_ROWS, _W), lambda i: (i, 0))],
        out_specs=pl.BlockSpec((_BLOCK_ROWS, _W), lambda i: (i, 0)),
        out_shape=jax.ShapeDtypeStruct((_ROWS, _W), jnp.float32),
    )(x)
    return out.reshape(_C, _H, _W)
